# Initial kernel scaffold; baseline (speedup 1.0000x reference)
import jax
import jax.numpy as jnp
from jax.experimental import pallas as pl


def kernel(input, fft_attrib_real, fft_attrib_imag, expl_p_real, expl_p_imag, k):
    raise NotImplementedError("write your pallas kernel here")


# speedup vs baseline: 24.5320x; 24.5320x over previous
import functools

import jax
import jax.numpy as jnp
from jax import lax
from jax.experimental import pallas as pl
from jax.experimental.pallas import tpu as pltpu
from jax.experimental.pallas import tpu_sc as plsc

_THRESHOLD = 0.001
_B, _C, _F = 128, 32, 2049


def _argmax_body(real_ref, imag_ref, vr_ref, fr_ref, vi_ref, fi_ref):
    b = pl.program_id(0)
    row_base = (b * _C + lax.broadcasted_iota(jnp.int32, (1, _C), 1)) * _F

    def one(x_ref, v_ref, f_ref):
        x = jnp.abs(x_ref[0])
        m = jnp.max(x, axis=0, keepdims=True)
        fidx = lax.broadcasted_iota(jnp.int32, x.shape, 0)
        idx = jnp.min(jnp.where(x == m, fidx, _F), axis=0, keepdims=True)
        v_ref[0] = m
        f_ref[0] = row_base + idx

    one(real_ref, vr_ref, fr_ref)
    one(imag_ref, vi_ref, fi_ref)


def _run_argmax(expl_p_real, expl_p_imag):
    out3 = jax.ShapeDtypeStruct((_B, 1, _C), jnp.float32)
    out3i = jax.ShapeDtypeStruct((_B, 1, _C), jnp.int32)
    in_spec = pl.BlockSpec((1, _F, _C), lambda b: (b, 0, 0))
    out_spec = pl.BlockSpec((1, 1, _C), lambda b: (b, 0, 0))
    return pl.pallas_call(
        _argmax_body,
        grid=(_B,),
        in_specs=[in_spec, in_spec],
        out_specs=[out_spec, out_spec, out_spec, out_spec],
        out_shape=[out3, out3i, out3, out3i],
    )(expl_p_real, expl_p_imag)


def _make_gather():
    info = plsc.get_sparse_core_info()
    nw = info.num_cores * info.num_subcores
    chunk = (_B * _C) // nw
    mesh = plsc.VectorSubcoreMesh(core_axis_name="c", subcore_axis_name="s")

    @functools.partial(
        pl.kernel,
        mesh=mesh,
        out_type=[jax.ShapeDtypeStruct((_B * _C,), jnp.float32)] * 2,
        scratch_types=[
            pltpu.VMEM((chunk,), jnp.int32),
            pltpu.VMEM((chunk,), jnp.float32),
            pltpu.VMEM((chunk,), jnp.int32),
            pltpu.VMEM((chunk,), jnp.float32),
            pltpu.SemaphoreType.DMA,
            pltpu.SemaphoreType.DMA,
        ],
    )
    def gather(fr_hbm, fi_hbm, real_hbm, imag_hbm, gr_hbm, gi_hbm,
               idxr_v, valr_v, idxi_v, vali_v, semr, semi):
        wid = lax.axis_index("s") * info.num_cores + lax.axis_index("c")
        base = wid * chunk
        pltpu.sync_copy(fr_hbm.at[pl.ds(base, chunk)], idxr_v)
        pltpu.sync_copy(fi_hbm.at[pl.ds(base, chunk)], idxi_v)
        cr = pltpu.async_copy(real_hbm.at[idxr_v], valr_v, semr)
        ci = pltpu.async_copy(imag_hbm.at[idxi_v], vali_v, semi)
        cr.wait()
        ci.wait()
        pltpu.sync_copy(valr_v, gr_hbm.at[pl.ds(base, chunk)])
        pltpu.sync_copy(vali_v, gi_hbm.at[pl.ds(base, chunk)])

    return gather


def _finish_body(vr_ref, vi_ref, gr_ref, gi_ref, out_ref):
    vr = vr_ref[:, 0, :]
    vi = vi_ref[:, 0, :]
    keep_r = jnp.min(vr, axis=1, keepdims=True) >= _THRESHOLD
    keep_i = jnp.min(vi, axis=1, keepdims=True) >= _THRESHOLD
    keep_b = jnp.logical_and(keep_r, keep_i)
    n_r = jnp.sum(keep_r.astype(jnp.float32))
    n_i = jnp.sum(keep_i.astype(jnp.float32))
    n_b = jnp.sum(keep_b.astype(jnp.float32))
    real_sum = jnp.sum(jnp.where(keep_r, gr_ref[...] ** 2, 0.0))
    imag_sum = jnp.sum(jnp.where(keep_b, gi_ref[...] ** 2, 0.0))
    real_loss = jnp.where(n_r > 0, real_sum / (n_r * _C) / n_r, 0.0)
    imag_loss = jnp.where((n_i > 0) & (n_b > 0),
                          imag_sum / (n_b * _C) / n_b, 0.0)
    out_ref[0, 0] = real_loss + imag_loss


def _run_finish(vr, vi, gr, gi):
    return pl.pallas_call(
        _finish_body,
        out_specs=pl.BlockSpec(memory_space=pltpu.SMEM),
        out_shape=jax.ShapeDtypeStruct((1, 1), jnp.float32),
    )(vr, vi, gr, gi)


def kernel(input, fft_attrib_real, fft_attrib_imag, expl_p_real, expl_p_imag, k):
    del input
    vr, fr, vi, fi = _run_argmax(expl_p_real, expl_p_imag)
    gather = _make_gather()
    gr, gi = gather(
        fr.reshape(_B * _C),
        fi.reshape(_B * _C),
        fft_attrib_real.reshape(_B * _C * _F),
        fft_attrib_imag.reshape(_B * _C * _F),
    )
    out = _run_finish(vr, vi, gr.reshape(_B, _C), gi.reshape(_B, _C))
    return out[0, 0] + 0.0 * jnp.asarray(k, dtype=jnp.float32)


# speedup vs baseline: 27.1930x; 1.1085x over previous
import functools

import jax
import jax.numpy as jnp
from jax import lax
from jax.experimental import pallas as pl
from jax.experimental.pallas import tpu as pltpu
from jax.experimental.pallas import tpu_sc as plsc

_THRESHOLD = 0.001
_B, _C, _F = 128, 32, 2049
_NC, _NS, _L = 2, 16, 16
_NW = _NC * _NS
_BPW = _B // _NW
_CHUNK = 512
_VMASK = 0x7FFFF000
_IMASK = 0xFFF


def _sc_body(er_hbm, ei_hbm, fr_hbm, fi_hbm, stats_hbm, ebuf, gbuf, obuf):
    wid = lax.axis_index("s") * _NC + lax.axis_index("c")
    lane = lax.iota(jnp.int32, _L)

    for bloc in range(_BPW):
        b = wid * _BPW + bloc

        packed = []
        for e_hbm in (er_hbm, ei_hbm):
            best = (jnp.zeros((_L,), jnp.int32), jnp.zeros((_L,), jnp.int32))

            for ci in range(4):
                pltpu.sync_copy(e_hbm.at[b, pl.ds(ci * _CHUNK, _CHUNK), :], ebuf)
                f_base = ci * _CHUNK

                def chunk_body(i, carry, f_base=f_base):
                    b0, b1 = carry
                    for j in range(16):
                        r = i * 16 + j
                        low = jnp.full((_L,), 4095 - (f_base + r), jnp.int32)
                        x0 = lax.bitcast_convert_type(ebuf[r, pl.ds(0, _L)], jnp.int32)
                        b0 = jnp.maximum(b0, (x0 & _VMASK) | low)
                        x1 = lax.bitcast_convert_type(ebuf[r, pl.ds(_L, _L)], jnp.int32)
                        b1 = jnp.maximum(b1, (x1 & _VMASK) | low)
                    return b0, b1

                best = lax.fori_loop(0, _CHUNK // 16, chunk_body, best)

            pltpu.sync_copy(e_hbm.at[b, pl.ds(_F - 1, 1), :], ebuf.at[pl.ds(0, 1), :])
            low = jnp.full((_L,), 4095 - (_F - 1), jnp.int32)
            x0 = lax.bitcast_convert_type(ebuf[0, pl.ds(0, _L)], jnp.int32)
            x1 = lax.bitcast_convert_type(ebuf[0, pl.ds(_L, _L)], jnp.int32)
            best = (jnp.maximum(best[0], (x0 & _VMASK) | low),
                    jnp.maximum(best[1], (x1 & _VMASK) | low))
            packed.append(best)

        for ai in range(2):
            v0 = lax.bitcast_convert_type(packed[ai][0] & _VMASK, jnp.float32)
            v1 = lax.bitcast_convert_type(packed[ai][1] & _VMASK, jnp.float32)
            obuf[pl.ds(128 + ai * 64 + bloc * _L, _L)] = jnp.minimum(v0, v1)

        for ai, f_hbm in enumerate((fr_hbm, fi_hbm)):
            acc = jnp.zeros((_L,), jnp.float32)
            for h in range(2):
                idx = 4095 - (packed[ai][h] & _IMASK)
                pltpu.sync_copy(f_hbm.at[b, pl.ds(h * _L, _L), :], gbuf)
                g = plsc.load_gather(gbuf, [lane, idx])
                acc = acc + g * g
            obuf[pl.ds(ai * 64 + bloc * _L, _L)] = acc

    pltpu.sync_copy(obuf, stats_hbm.at[wid])


def _run_sc(expl_p_real, expl_p_imag, fft_attrib_real, fft_attrib_imag):
    mesh = plsc.VectorSubcoreMesh(core_axis_name="c", subcore_axis_name="s")
    run = functools.partial(
        pl.kernel,
        mesh=mesh,
        out_type=jax.ShapeDtypeStruct((_NW, 256), jnp.float32),
        scratch_types=[
            pltpu.VMEM((_CHUNK, _C), jnp.float32),
            pltpu.VMEM((_L, _F), jnp.float32),
            pltpu.VMEM((256,), jnp.float32),
        ],
        compiler_params=pltpu.CompilerParams(use_tc_tiling_on_sc=False,
                                             needs_layout_passes=False),
    )(_sc_body)
    return run(expl_p_real, expl_p_imag, fft_attrib_real, fft_attrib_imag)


def _finish_body(stats_ref, out_ref):
    accr = stats_ref[:, 0:64].reshape(_NW, _BPW, _L)
    acci = stats_ref[:, 64:128].reshape(_NW, _BPW, _L)
    vminr = stats_ref[:, 128:192].reshape(_NW, _BPW, _L)
    vmini = stats_ref[:, 192:256].reshape(_NW, _BPW, _L)
    bsum_r = jnp.sum(accr, axis=2)
    bsum_i = jnp.sum(acci, axis=2)
    keep_r = jnp.min(vminr, axis=2) >= _THRESHOLD
    keep_i = jnp.min(vmini, axis=2) >= _THRESHOLD
    keep_b = jnp.logical_and(keep_r, keep_i)
    n_r = jnp.sum(keep_r.astype(jnp.float32))
    n_i = jnp.sum(keep_i.astype(jnp.float32))
    n_b = jnp.sum(keep_b.astype(jnp.float32))
    real_sum = jnp.sum(jnp.where(keep_r, bsum_r, 0.0))
    imag_sum = jnp.sum(jnp.where(keep_b, bsum_i, 0.0))
    real_loss = jnp.where(n_r > 0, real_sum / (n_r * _C) / n_r, 0.0)
    imag_loss = jnp.where((n_i > 0) & (n_b > 0),
                          imag_sum / (n_b * _C) / n_b, 0.0)
    out_ref[0, 0] = real_loss + imag_loss


def _run_finish(stats):
    return pl.pallas_call(
        _finish_body,
        out_specs=pl.BlockSpec(memory_space=pltpu.SMEM),
        out_shape=jax.ShapeDtypeStruct((1, 1), jnp.float32),
    )(stats)


def kernel(input, fft_attrib_real, fft_attrib_imag, expl_p_real, expl_p_imag, k):
    del input
    stats = _run_sc(expl_p_real, expl_p_imag, fft_attrib_real, fft_attrib_imag)
    out = _run_finish(stats)
    return out[0, 0] + 0.0 * jnp.asarray(k, dtype=jnp.float32)


# speedup vs baseline: 28.6372x; 1.0531x over previous
import functools

import jax
import jax.numpy as jnp
from jax import lax
from jax.experimental import pallas as pl
from jax.experimental.pallas import tpu as pltpu
from jax.experimental.pallas import tpu_sc as plsc

_THRESHOLD = 0.001
_B, _C, _F = 128, 32, 2049
_NC, _NS, _L = 2, 16, 16
_NW = _NC * _NS
_BPW = _B // _NW
_CHUNK = 512
_VMASK = 0x7FFFF000
_IMASK = 0xFFF


def _sc_body(er_hbm, ei_hbm, fr_hbm, fi_hbm, stats_hbm, ebuf, gbuf, obuf):
    wid = lax.axis_index("s") * _NC + lax.axis_index("c")
    lane = lax.iota(jnp.int32, _L)

    for bloc in range(_BPW):
        b = wid * _BPW + bloc

        packed = []
        for e_hbm in (er_hbm, ei_hbm):
            best = (jnp.zeros((_L,), jnp.int32), jnp.zeros((_L,), jnp.int32))

            for ci in range(4):
                pltpu.sync_copy(e_hbm.at[b, pl.ds(ci * _CHUNK, _CHUNK), :], ebuf)
                f_base = ci * _CHUNK

                def chunk_body(i, carry, f_base=f_base):
                    b0, b1 = carry
                    for j in range(16):
                        r = i * 16 + j
                        low = jnp.full((_L,), 4095 - (f_base + r), jnp.int32)
                        x0 = lax.bitcast_convert_type(ebuf[r, pl.ds(0, _L)], jnp.int32)
                        b0 = jnp.maximum(b0, (x0 & _VMASK) | low)
                        x1 = lax.bitcast_convert_type(ebuf[r, pl.ds(_L, _L)], jnp.int32)
                        b1 = jnp.maximum(b1, (x1 & _VMASK) | low)
                    return b0, b1

                best = lax.fori_loop(0, _CHUNK // 16, chunk_body, best)

            pltpu.sync_copy(e_hbm.at[b, pl.ds(_F - 1, 1), :], ebuf.at[pl.ds(0, 1), :])
            low = jnp.full((_L,), 4095 - (_F - 1), jnp.int32)
            x0 = lax.bitcast_convert_type(ebuf[0, pl.ds(0, _L)], jnp.int32)
            x1 = lax.bitcast_convert_type(ebuf[0, pl.ds(_L, _L)], jnp.int32)
            best = (jnp.maximum(best[0], (x0 & _VMASK) | low),
                    jnp.maximum(best[1], (x1 & _VMASK) | low))
            packed.append(best)

        for ai in range(2):
            v0 = lax.bitcast_convert_type(packed[ai][0] & _VMASK, jnp.float32)
            v1 = lax.bitcast_convert_type(packed[ai][1] & _VMASK, jnp.float32)
            obuf[pl.ds(128 + ai * 64 + bloc * _L, _L)] = jnp.minimum(v0, v1)

        for ai, f_hbm in enumerate((fr_hbm, fi_hbm)):
            acc = jnp.zeros((_L,), jnp.float32)
            for h in range(2):
                idx = 4095 - (packed[ai][h] & _IMASK)
                pltpu.sync_copy(f_hbm.at[b, pl.ds(h * _L, _L), :], gbuf)
                g = plsc.load_gather(gbuf, [lane, idx])
                acc = acc + g * g
            obuf[pl.ds(ai * 64 + bloc * _L, _L)] = acc

    pltpu.sync_copy(obuf, stats_hbm.at[pl.ds(wid * 256, 256)])


def _run_sc(expl_p_real, expl_p_imag, fft_attrib_real, fft_attrib_imag):
    mesh = plsc.VectorSubcoreMesh(core_axis_name="c", subcore_axis_name="s")
    run = functools.partial(
        pl.kernel,
        mesh=mesh,
        out_type=jax.ShapeDtypeStruct((_NW * 256,), jnp.float32),
        scratch_types=[
            pltpu.VMEM((_CHUNK, _C), jnp.float32),
            pltpu.VMEM((_L, _F), jnp.float32),
            pltpu.VMEM((256,), jnp.float32),
        ],
        compiler_params=pltpu.CompilerParams(use_tc_tiling_on_sc=True,
                                             needs_layout_passes=False),
    )(_sc_body)
    return run(expl_p_real, expl_p_imag, fft_attrib_real, fft_attrib_imag)


def _finish_body(stats_ref, out_ref):
    acc = stats_ref[:, 0, :]
    vmin = stats_ref[:, 1, :]
    accr = acc[:, 0:64].reshape(_NW, _BPW, _L)
    acci = acc[:, 64:128].reshape(_NW, _BPW, _L)
    vminr = vmin[:, 0:64].reshape(_NW, _BPW, _L)
    vmini = vmin[:, 64:128].reshape(_NW, _BPW, _L)
    bsum_r = jnp.sum(accr, axis=2)
    bsum_i = jnp.sum(acci, axis=2)
    keep_r = jnp.min(vminr, axis=2) >= _THRESHOLD
    keep_i = jnp.min(vmini, axis=2) >= _THRESHOLD
    keep_b = jnp.logical_and(keep_r, keep_i)
    n_r = jnp.sum(keep_r.astype(jnp.float32))
    n_i = jnp.sum(keep_i.astype(jnp.float32))
    n_b = jnp.sum(keep_b.astype(jnp.float32))
    real_sum = jnp.sum(jnp.where(keep_r, bsum_r, 0.0))
    imag_sum = jnp.sum(jnp.where(keep_b, bsum_i, 0.0))
    real_loss = jnp.where(n_r > 0, real_sum / (n_r * _C) / n_r, 0.0)
    imag_loss = jnp.where((n_i > 0) & (n_b > 0),
                          imag_sum / (n_b * _C) / n_b, 0.0)
    out_ref[0, 0] = real_loss + imag_loss


def _run_finish(stats):
    return pl.pallas_call(
        _finish_body,
        out_specs=pl.BlockSpec(memory_space=pltpu.SMEM),
        out_shape=jax.ShapeDtypeStruct((1, 1), jnp.float32),
    )(stats)


def kernel(input, fft_attrib_real, fft_attrib_imag, expl_p_real, expl_p_imag, k):
    del input
    stats = _run_sc(expl_p_real, expl_p_imag, fft_attrib_real, fft_attrib_imag)
    out = _run_finish(stats.reshape(_NW, 2, 128))
    return out[0, 0] + 0.0 * jnp.asarray(k, dtype=jnp.float32)
